# baseline (speedup 1.0000x reference)
import jax
import jax.numpy as jnp
from jax.experimental import pallas as pl
from jax.experimental.pallas import tpu as pltpu

_LAMBDA_COORD = 5.0
_LAMBDA_NOOBJ = 0.5
_C = 80
_S = 4096
_B = 32
_A = 3


def _group_loss(p, t):
    f32 = jnp.float32
    p0 = p[0:1, :]
    p1 = p[1:2, :]
    p2 = p[2:3, :]
    p3 = p[3:4, :]
    z = p[4:5, :]
    cs = p[5:, :]

    t0 = t[0:1, :]
    t1 = t[1:2, :]
    t2r = t[2:3, :]
    t3 = t[3:4, :]
    t4 = t[4:5, :]
    tc = t[5:, :]

    o = (t4 == 1.0).astype(f32)
    nb = (t4 == 0.0).astype(f32)

    sx = 1.0 / (1.0 + jnp.exp(-p0))
    sy = 1.0 / (1.0 + jnp.exp(-p1))
    dx = sx - t0
    dy = sy - t1
    dw = p2 - t2r
    dh = p3 - t3
    loc_row = o * (dx * dx + dy * dy + dw * dw + dh * dh)

    g = jnp.maximum(z, 0.0) + jnp.log(1.0 + jnp.exp(-jnp.abs(z)))
    conf_row = o * (g - z) + _LAMBDA_NOOBJ * (nb * g)

    mx = jnp.max(cs, axis=0, keepdims=True)
    lse = mx + jnp.log(jnp.sum(jnp.exp(cs - mx), axis=0, keepdims=True))

    maxv = jnp.max(tc, axis=0, keepdims=True)
    iota_c = jax.lax.broadcasted_iota(jnp.int32, tc.shape, 0)
    cand = jnp.where(tc == maxv, iota_c, _C)
    idxm = jnp.min(cand, axis=0, keepdims=True)
    picked = jnp.sum(jnp.where(cand == idxm, cs, 0.0), axis=0,
                     keepdims=True)
    cls_row = o * (lse - picked)

    total_row = _LAMBDA_COORD * loc_row + conf_row + cls_row
    return jnp.sum(total_row, axis=1, keepdims=True)


_BB = 2


def _body(pred_a_ref, pred_b_ref, targ_a_ref, targ_b_ref, out_ref):
    acc = jnp.zeros((1, 1), jnp.float32)
    for b in range(_BB):
        for a in range(2):
            p = pred_a_ref[b, a]
            t = jnp.swapaxes(targ_a_ref[b, a].reshape(_S, 85), 0, 1)
            acc = acc + _group_loss(p, t)
        p = pred_b_ref[b, 0]
        t = jnp.swapaxes(targ_b_ref[b, 0].reshape(_S, 85), 0, 1)
        acc = acc + _group_loss(p, t)

    @pl.when(pl.program_id(0) == 0)
    def _():
        out_ref[...] = jnp.zeros_like(out_ref)

    out_ref[...] += acc * (1.0 / _B)


def kernel(predictions, targets):
    pred_r = predictions.reshape(_B, _A, 85, _S)
    out = pl.pallas_call(
        _body,
        grid=(_B // _BB,),
        in_specs=[
            pl.BlockSpec((_BB, 2, 85, _S), lambda b: (b, 0, 0, 0)),
            pl.BlockSpec((_BB, 1, 85, _S), lambda b: (b, 2, 0, 0)),
            pl.BlockSpec((_BB, 2, 64, 64, 85), lambda b: (b, 0, 0, 0, 0)),
            pl.BlockSpec((_BB, 1, 64, 64, 85), lambda b: (b, 2, 0, 0, 0)),
        ],
        out_specs=pl.BlockSpec((1, 1), lambda b: (0, 0)),
        out_shape=jax.ShapeDtypeStruct((1, 1), jnp.float32),
        compiler_params=pltpu.CompilerParams(
            dimension_semantics=("arbitrary",)),
    )(pred_r, pred_r, targets, targets)
    return out[0, 0]


# speedup vs baseline: 2.6559x; 2.6559x over previous
import jax
import jax.numpy as jnp
from jax.experimental import pallas as pl
from jax.experimental.pallas import tpu as pltpu

_LAMBDA_COORD = 5.0
_LAMBDA_NOOBJ = 0.5
_C = 80
_S = 4096
_B = 32
_A = 3


def _group_loss(p, t):
    f32 = jnp.float32
    p0 = p[0:1, :]
    p1 = p[1:2, :]
    p2 = p[2:3, :]
    p3 = p[3:4, :]
    z = p[4:5, :]
    cs = p[5:, :]

    t0 = t[0:1, :]
    t1 = t[1:2, :]
    t2r = t[2:3, :]
    t3 = t[3:4, :]
    t4 = t[4:5, :]
    tc = t[5:, :]

    o = (t4 == 1.0).astype(f32)
    nb = (t4 == 0.0).astype(f32)

    sx = 1.0 / (1.0 + jnp.exp(-p0))
    sy = 1.0 / (1.0 + jnp.exp(-p1))
    dx = sx - t0
    dy = sy - t1
    dw = p2 - t2r
    dh = p3 - t3
    loc_row = o * (dx * dx + dy * dy + dw * dw + dh * dh)

    g = jnp.maximum(z, 0.0) + jnp.log(1.0 + jnp.exp(-jnp.abs(z)))
    conf_row = o * (g - z) + _LAMBDA_NOOBJ * (nb * g)

    mx = jnp.max(cs, axis=0, keepdims=True)
    lse = mx + jnp.log(jnp.sum(jnp.exp(cs - mx), axis=0, keepdims=True))

    maxv = jnp.max(tc, axis=0, keepdims=True)
    iota_c = jax.lax.broadcasted_iota(jnp.int32, tc.shape, 0)
    cand = jnp.where(tc == maxv, iota_c, _C)
    idxm = jnp.min(cand, axis=0, keepdims=True)
    picked = jnp.sum(jnp.where(cand == idxm, cs, 0.0), axis=0,
                     keepdims=True)
    cls_row = o * (lse - picked)

    total_row = _LAMBDA_COORD * loc_row + conf_row + cls_row
    return jnp.sum(total_row, axis=1, keepdims=True)


_BB = 2


def _body(pred_ref, targ_ref, out_ref):
    acc = jnp.zeros((1, 1), jnp.float32)
    for b in range(_BB):
        for a in range(_A):
            p = pred_ref[b, 85 * a:85 * (a + 1), :]
            t = jnp.swapaxes(targ_ref[b, a].reshape(_S, 85), 0, 1)
            acc = acc + _group_loss(p, t)

    @pl.when(pl.program_id(0) == 0)
    def _():
        out_ref[...] = jnp.zeros_like(out_ref)

    out_ref[...] += acc * (1.0 / _B)


def kernel(predictions, targets):
    pred_r = predictions.reshape(_B, _A * 85, _S)
    out = pl.pallas_call(
        _body,
        grid=(_B // _BB,),
        in_specs=[
            pl.BlockSpec((_BB, _A * 85, _S), lambda b: (b, 0, 0)),
            pl.BlockSpec((_BB, _A, 64, 64, 85), lambda b: (b, 0, 0, 0, 0)),
        ],
        out_specs=pl.BlockSpec((1, 1), lambda b: (0, 0)),
        out_shape=jax.ShapeDtypeStruct((1, 1), jnp.float32),
        compiler_params=pltpu.CompilerParams(
            dimension_semantics=("arbitrary",)),
    )(pred_r, targets)
    return out[0, 0]
